# Initial kernel scaffold; baseline (speedup 1.0000x reference)
import functools

import jax
import jax.numpy as jnp
from jax import lax
from jax.experimental import pallas as pl
from jax.experimental.pallas import tpu as pltpu
from jax.experimental.pallas import tpu_sc as plsc

N = 10000
E = 320000
D = 128
AW = 144
NS = 16
EPT = E // NS
CH = 128
NFULL = EPT // CH
TAIL = EPT - NFULL * CH
RPT = N // NS
RB = 125

_ROWBLK = 1000


def _aug_pad(rows):
    col = lax.broadcasted_iota(jnp.int32, (rows, 16), 1)
    return jnp.where(col == 0, 1.0, 0.0).astype(jnp.float32)


def _proj_body(x_ref, w_ref, b_ref, o_ref):
    h = jnp.dot(x_ref[...], w_ref[...], preferred_element_type=jnp.float32,
                precision=lax.Precision.HIGHEST)
    h = jnp.maximum(h + b_ref[...][None, :], 0.0)
    o_ref[...] = jnp.concatenate([h, _aug_pad(h.shape[0])], axis=1)


def _proj(x, w, b):
    return pl.pallas_call(
        _proj_body,
        grid=(N // _ROWBLK,),
        in_specs=[
            pl.BlockSpec((_ROWBLK, D), lambda i: (i, 0)),
            pl.BlockSpec((D, D), lambda i: (0, 0)),
            pl.BlockSpec((D,), lambda i: (0,)),
        ],
        out_specs=pl.BlockSpec((_ROWBLK, AW), lambda i: (i, 0)),
        out_shape=jax.ShapeDtypeStruct((N, AW), jnp.float32),
    )(x, w, b)


def _combine_body(final, tab_ref, acc_ref, ws_ref, wn_ref, b_ref, o_ref):
    hd = tab_ref[:, :D]
    acc = acc_ref[...]
    ssum = acc[:, :D]
    deg = acc[:, D:D + 1]
    hn = ssum / jnp.maximum(deg, 1.0)
    z = (jnp.dot(hd, ws_ref[...], preferred_element_type=jnp.float32,
                 precision=lax.Precision.HIGHEST)
         + jnp.dot(hn, wn_ref[...], preferred_element_type=jnp.float32,
                   precision=lax.Precision.HIGHEST)
         + b_ref[...][None, :])
    z = jnp.maximum(z, 0.0)
    if final:
        nrm = jnp.sqrt(jnp.sum(z * z, axis=1, keepdims=True))
        o_ref[...] = z / jnp.maximum(nrm, 1e-12)
    else:
        o_ref[...] = jnp.concatenate([z, _aug_pad(z.shape[0])], axis=1)


def _combine(tab, acc, ws, wn, b, final):
    ow = D if final else AW
    return pl.pallas_call(
        functools.partial(_combine_body, final),
        grid=(N // _ROWBLK,),
        in_specs=[
            pl.BlockSpec((_ROWBLK, AW), lambda i: (i, 0)),
            pl.BlockSpec((_ROWBLK, AW), lambda i: (i, 0)),
            pl.BlockSpec((D, D), lambda i: (0, 0)),
            pl.BlockSpec((D, D), lambda i: (0, 0)),
            pl.BlockSpec((D,), lambda i: (0,)),
        ],
        out_specs=pl.BlockSpec((_ROWBLK, ow), lambda i: (i, 0)),
        out_shape=jax.ShapeDtypeStruct((N, ow), jnp.float32),
    )(tab, acc, ws, wn, b)


def _seg_body(tab_cb, tab_cl, src_cb, dst_cb, src_cl, dst_cl,
              acc_u_out, acc_i_out,
              src_v, dst_v, rows_v, src_t, dst_t, rows_t, acc_sh, sem):
    c = lax.axis_index("c")
    s = lax.axis_index("s")

    zv = jnp.zeros((16,), jnp.float32)

    def _zero_row(r, carry):
        for j in range(AW // 16):
            rows_v[r, pl.ds(j * 16, 16)] = zv
        return carry

    lax.fori_loop(0, RB, _zero_row, 0)

    def _run(tab, srcs, dsts, out):
        for k in range(RPT // RB):
            pltpu.sync_copy(rows_v.at[pl.ds(0, RB)],
                            acc_sh.at[pl.ds(s * RPT + k * RB, RB)])
        plsc.subcore_barrier()

        base0 = s * EPT

        def _step(i, carry):
            base = base0 + i * CH
            pltpu.sync_copy(srcs.at[pl.ds(base, CH)], src_v)
            pltpu.sync_copy(dsts.at[pl.ds(base, CH)], dst_v)
            pltpu.async_copy(tab.at[src_v], rows_v, sem).wait()
            pltpu.sync_copy(rows_v, acc_sh.at[dst_v], add=True)
            return carry

        lax.fori_loop(0, NFULL, _step, 0)

        if TAIL:
            tb = base0 + NFULL * CH
            pltpu.sync_copy(srcs.at[pl.ds(tb, TAIL)], src_t)
            pltpu.sync_copy(dsts.at[pl.ds(tb, TAIL)], dst_t)
            pltpu.async_copy(tab.at[src_t], rows_t, sem).wait()
            pltpu.sync_copy(rows_t, acc_sh.at[dst_t], add=True)

        plsc.subcore_barrier()
        pltpu.sync_copy(acc_sh.at[pl.ds(s * RPT, RPT)],
                        out.at[pl.ds(s * RPT, RPT)])

    @pl.when(c == 0)
    def _():
        _run(tab_cb, src_cb, dst_cb, acc_u_out)

    @pl.when(c == 1)
    def _():
        _run(tab_cl, src_cl, dst_cl, acc_i_out)


_seg_layer = pl.kernel(
    _seg_body,
    out_type=[
        jax.ShapeDtypeStruct((N, AW), jnp.float32),
        jax.ShapeDtypeStruct((N, AW), jnp.float32),
    ],
    mesh=plsc.VectorSubcoreMesh(core_axis_name="c", subcore_axis_name="s"),
    scratch_types=[
        pltpu.VMEM((CH,), jnp.int32),
        pltpu.VMEM((CH,), jnp.int32),
        pltpu.VMEM((CH, AW), jnp.float32),
        pltpu.VMEM((TAIL,), jnp.int32),
        pltpu.VMEM((TAIL,), jnp.int32),
        pltpu.VMEM((TAIL, AW), jnp.float32),
        pltpu.VMEM_SHARED((N, AW), jnp.float32),
        pltpu.SemaphoreType.DMA,
    ],
)


def kernel(x_user, x_item, ei_clicks, ei_clicked_by, Wp_user, bp_user,
           Wp_item, bp_item, Ws0_clicks, Wn0_clicks, b0_clicks, Ws0_cb,
           Wn0_cb, b0_cb, Ws1_clicks, Wn1_clicks, b1_clicks, Ws1_cb,
           Wn1_cb, b1_cb):
    src_cb = ei_clicked_by[0]
    dst_cb = ei_clicked_by[1]
    src_cl = ei_clicks[0]
    dst_cl = ei_clicks[1]

    hu = _proj(x_user, Wp_user, bp_user)
    hi = _proj(x_item, Wp_item, bp_item)

    acc_u, acc_i = _seg_layer(hi, hu, src_cb, dst_cb, src_cl, dst_cl)
    hu = _combine(hu, acc_u, Ws0_cb, Wn0_cb, b0_cb, final=False)
    hi = _combine(hi, acc_i, Ws0_clicks, Wn0_clicks, b0_clicks, final=False)

    acc_u, acc_i = _seg_layer(hi, hu, src_cb, dst_cb, src_cl, dst_cl)
    h_u = _combine(hu, acc_u, Ws1_cb, Wn1_cb, b1_cb, final=True)
    h_i = _combine(hi, acc_i, Ws1_clicks, Wn1_clicks, b1_clicks, final=True)
    return (h_u, h_i)


# speedup vs baseline: 4.0124x; 4.0124x over previous
import functools

import jax
import jax.numpy as jnp
from jax import lax
from jax.experimental import pallas as pl
from jax.experimental.pallas import tpu as pltpu
from jax.experimental.pallas import tpu_sc as plsc

N = 10000
E = 320000
D = 128
NS = 16
EPT = E // NS
CH = 128
NFULL = EPT // CH
TAIL = EPT - NFULL * CH
NP = 10240
RPT = NP // NS
RB = 128

_ROWBLK = 1000


def _proj_body(x_ref, w_ref, b_ref, o_ref):
    h = jnp.dot(x_ref[...], w_ref[...], preferred_element_type=jnp.float32,
                precision=lax.Precision.HIGHEST)
    o_ref[...] = jnp.maximum(h + b_ref[...][None, :], 0.0)


def _proj(x, w, b):
    return pl.pallas_call(
        _proj_body,
        grid=(N // _ROWBLK,),
        in_specs=[
            pl.BlockSpec((_ROWBLK, D), lambda i: (i, 0)),
            pl.BlockSpec((D, D), lambda i: (0, 0)),
            pl.BlockSpec((D,), lambda i: (0,)),
        ],
        out_specs=pl.BlockSpec((_ROWBLK, D), lambda i: (i, 0)),
        out_shape=jax.ShapeDtypeStruct((N, D), jnp.float32),
    )(x, w, b)


def _combine_body(final, tab_ref, acc_ref, degp_ref, ws_ref, wn_ref, b_ref,
                  o_ref):
    deg = jnp.sum(degp_ref[...], axis=1)
    hn = acc_ref[...] / jnp.maximum(deg, 1.0)[:, None]
    z = (jnp.dot(tab_ref[...], ws_ref[...], preferred_element_type=jnp.float32,
                 precision=lax.Precision.HIGHEST)
         + jnp.dot(hn, wn_ref[...], preferred_element_type=jnp.float32,
                   precision=lax.Precision.HIGHEST)
         + b_ref[...][None, :])
    z = jnp.maximum(z, 0.0)
    if final:
        nrm = jnp.sqrt(jnp.sum(z * z, axis=1, keepdims=True))
        z = z / jnp.maximum(nrm, 1e-12)
    o_ref[...] = z


def _combine(tab, acc, degp, ws, wn, b, final):
    return pl.pallas_call(
        functools.partial(_combine_body, final),
        grid=(N // _ROWBLK,),
        in_specs=[
            pl.BlockSpec((_ROWBLK, D), lambda i: (i, 0)),
            pl.BlockSpec((_ROWBLK, D), lambda i: (i, 0)),
            pl.BlockSpec((_ROWBLK, NS), lambda i: (i, 0)),
            pl.BlockSpec((D, D), lambda i: (0, 0)),
            pl.BlockSpec((D, D), lambda i: (0, 0)),
            pl.BlockSpec((D,), lambda i: (0,)),
        ],
        out_specs=pl.BlockSpec((_ROWBLK, D), lambda i: (i, 0)),
        out_shape=jax.ShapeDtypeStruct((N, D), jnp.float32),
    )(tab, acc, degp, ws, wn, b)


def _deg_body(dst_cb, dst_cl, degp_u_out, degp_i_out, idx_v, deg_local):
    c = lax.axis_index("c")
    s = lax.axis_index("s")
    zv = jnp.zeros((16,), jnp.float32)
    ones = jnp.ones((16,), jnp.float32)

    def _zero(i, carry):
        deg_local[pl.ds(i * 16, 16)] = zv
        return carry

    lax.fori_loop(0, NP // 16, _zero, 0)

    def _run(dsts, out):
        base0 = s * EPT

        def _step(i, carry):
            pltpu.sync_copy(dsts.at[pl.ds(base0 + i * CH, CH)], idx_v)
            for k in range(CH // 16):
                idx16 = idx_v[pl.ds(k * 16, 16)]
                plsc.addupdate_scatter(deg_local, [idx16], ones)
            return carry

        lax.fori_loop(0, NFULL, _step, 0)

        if TAIL:
            pltpu.sync_copy(dsts.at[pl.ds(base0 + NFULL * CH, TAIL)],
                            idx_v.at[pl.ds(0, TAIL)])
            for k in range(TAIL // 16):
                idx16 = idx_v[pl.ds(k * 16, 16)]
                plsc.addupdate_scatter(deg_local, [idx16], ones)

        pltpu.sync_copy(deg_local, out.at[pl.ds(s * NP, NP)])

    @pl.when(c == 0)
    def _():
        _run(dst_cb, degp_u_out)

    @pl.when(c == 1)
    def _():
        _run(dst_cl, degp_i_out)


@functools.lru_cache(maxsize=None)
def _deg_kernel():
    return pl.kernel(
        _deg_body,
        out_type=[
            jax.ShapeDtypeStruct((NS * NP,), jnp.float32),
            jax.ShapeDtypeStruct((NS * NP,), jnp.float32),
        ],
        mesh=plsc.VectorSubcoreMesh(core_axis_name="c", subcore_axis_name="s"),
        scratch_types=[
            pltpu.VMEM((CH,), jnp.int32),
            pltpu.VMEM((NP,), jnp.float32),
        ],
        compiler_params=pltpu.CompilerParams(needs_layout_passes=False),
    )


def _degrees(dst_cb, dst_cl):
    degp_u, degp_i = _deg_kernel()(dst_cb, dst_cl)
    return (degp_u.reshape(NS, NP)[:, :N].T, degp_i.reshape(NS, NP)[:, :N].T)


def _seg_body(tab_cb, tab_cl, src_cb, dst_cb, src_cl, dst_cl,
              acc_u_out, acc_i_out,
              src_v, dst_v, rows_v, src_t, dst_t, rows_t, acc_sh, sem):
    c = lax.axis_index("c")
    s = lax.axis_index("s")

    zv = jnp.zeros((16,), jnp.float32)

    def _zero_row(r, carry):
        for j in range(D // 16):
            rows_v[r, pl.ds(j * 16, 16)] = zv
        return carry

    lax.fori_loop(0, RB, _zero_row, 0)

    def _run(tab, srcs, dsts, out):
        for k in range(RPT // RB):
            pltpu.sync_copy(rows_v,
                            acc_sh.at[pl.ds(s * RPT + k * RB, RB)])
        plsc.subcore_barrier()

        base0 = s * EPT

        def _step(i, carry):
            base = base0 + i * CH
            pltpu.sync_copy(srcs.at[pl.ds(base, CH)], src_v)
            pltpu.sync_copy(dsts.at[pl.ds(base, CH)], dst_v)
            pltpu.async_copy(tab.at[src_v], rows_v, sem).wait()
            pltpu.sync_copy(rows_v, acc_sh.at[dst_v], add=True)
            return carry

        lax.fori_loop(0, NFULL, _step, 0)

        if TAIL:
            tb = base0 + NFULL * CH
            pltpu.sync_copy(srcs.at[pl.ds(tb, TAIL)], src_t)
            pltpu.sync_copy(dsts.at[pl.ds(tb, TAIL)], dst_t)
            pltpu.async_copy(tab.at[src_t], rows_t, sem).wait()
            pltpu.sync_copy(rows_t, acc_sh.at[dst_t], add=True)

        plsc.subcore_barrier()
        pltpu.sync_copy(acc_sh.at[pl.ds(s * RPT, RPT)],
                        out.at[pl.ds(s * RPT, RPT)])

    @pl.when(c == 0)
    def _():
        _run(tab_cb, src_cb, dst_cb, acc_u_out)

    @pl.when(c == 1)
    def _():
        _run(tab_cl, src_cl, dst_cl, acc_i_out)


@functools.lru_cache(maxsize=None)
def _seg_layer_kernel():
    return pl.kernel(
        _seg_body,
        out_type=[
            jax.ShapeDtypeStruct((NP, D), jnp.float32),
            jax.ShapeDtypeStruct((NP, D), jnp.float32),
        ],
        mesh=plsc.VectorSubcoreMesh(core_axis_name="c", subcore_axis_name="s"),
        scratch_types=[
            pltpu.VMEM((CH,), jnp.int32),
            pltpu.VMEM((CH,), jnp.int32),
            pltpu.VMEM((CH, D), jnp.float32),
            pltpu.VMEM((TAIL,), jnp.int32),
            pltpu.VMEM((TAIL,), jnp.int32),
            pltpu.VMEM((TAIL, D), jnp.float32),
            pltpu.VMEM_SHARED((NP, D), jnp.float32),
            pltpu.SemaphoreType.DMA,
        ],
    )


def _seg_layer(tab_cb, tab_cl, src_cb, dst_cb, src_cl, dst_cl):
    acc_u, acc_i = _seg_layer_kernel()(tab_cb, tab_cl, src_cb, dst_cb,
                                       src_cl, dst_cl)
    return acc_u[:N], acc_i[:N]


def kernel(x_user, x_item, ei_clicks, ei_clicked_by, Wp_user, bp_user,
           Wp_item, bp_item, Ws0_clicks, Wn0_clicks, b0_clicks, Ws0_cb,
           Wn0_cb, b0_cb, Ws1_clicks, Wn1_clicks, b1_clicks, Ws1_cb,
           Wn1_cb, b1_cb):
    src_cb = ei_clicked_by[0]
    dst_cb = ei_clicked_by[1]
    src_cl = ei_clicks[0]
    dst_cl = ei_clicks[1]

    degp_u, degp_i = _degrees(dst_cb, dst_cl)

    hu = _proj(x_user, Wp_user, bp_user)
    hi = _proj(x_item, Wp_item, bp_item)

    acc_u, acc_i = _seg_layer(hi, hu, src_cb, dst_cb, src_cl, dst_cl)
    hu = _combine(hu, acc_u, degp_u, Ws0_cb, Wn0_cb, b0_cb, final=False)
    hi = _combine(hi, acc_i, degp_i, Ws0_clicks, Wn0_clicks, b0_clicks,
                  final=False)

    acc_u, acc_i = _seg_layer(hi, hu, src_cb, dst_cb, src_cl, dst_cl)
    h_u = _combine(hu, acc_u, degp_u, Ws1_cb, Wn1_cb, b1_cb, final=True)
    h_i = _combine(hi, acc_i, degp_i, Ws1_clicks, Wn1_clicks, b1_clicks,
                   final=True)
    return (h_u, h_i)


# speedup vs baseline: 5.7515x; 1.4334x over previous
import functools

import jax
import jax.numpy as jnp
from jax import lax
from jax.experimental import pallas as pl
from jax.experimental.pallas import tpu as pltpu
from jax.experimental.pallas import tpu_sc as plsc

N = 10000
E = 320000
D = 128
NS = 16
EPT = E // NS
CH = 64
NFULL = EPT // CH
TAIL = EPT - NFULL * CH
NP = 10240
RPT = NP // NS
RB = CH

_ROWBLK = 1000


def _proj_body(x_ref, w_ref, b_ref, o_ref):
    h = jnp.dot(x_ref[...], w_ref[...], preferred_element_type=jnp.float32,
                precision=lax.Precision.HIGHEST)
    o_ref[...] = jnp.maximum(h + b_ref[...][None, :], 0.0)


def _proj(x, w, b):
    return pl.pallas_call(
        _proj_body,
        grid=(N // _ROWBLK,),
        in_specs=[
            pl.BlockSpec((_ROWBLK, D), lambda i: (i, 0)),
            pl.BlockSpec((D, D), lambda i: (0, 0)),
            pl.BlockSpec((D,), lambda i: (0,)),
        ],
        out_specs=pl.BlockSpec((_ROWBLK, D), lambda i: (i, 0)),
        out_shape=jax.ShapeDtypeStruct((N, D), jnp.float32),
    )(x, w, b)


def _combine_body(final, tab_ref, acc_ref, degp_ref, ws_ref, wn_ref, b_ref,
                  o_ref):
    deg = jnp.sum(degp_ref[...], axis=1)
    hn = acc_ref[...] / jnp.maximum(deg, 1.0)[:, None]
    z = (jnp.dot(tab_ref[...], ws_ref[...], preferred_element_type=jnp.float32,
                 precision=lax.Precision.HIGHEST)
         + jnp.dot(hn, wn_ref[...], preferred_element_type=jnp.float32,
                   precision=lax.Precision.HIGHEST)
         + b_ref[...][None, :])
    z = jnp.maximum(z, 0.0)
    if final:
        nrm = jnp.sqrt(jnp.sum(z * z, axis=1, keepdims=True))
        z = z / jnp.maximum(nrm, 1e-12)
    o_ref[...] = z


def _combine(tab, acc, degp, ws, wn, b, final):
    return pl.pallas_call(
        functools.partial(_combine_body, final),
        grid=(N // _ROWBLK,),
        in_specs=[
            pl.BlockSpec((_ROWBLK, D), lambda i: (i, 0)),
            pl.BlockSpec((_ROWBLK, D), lambda i: (i, 0)),
            pl.BlockSpec((_ROWBLK, NS), lambda i: (i, 0)),
            pl.BlockSpec((D, D), lambda i: (0, 0)),
            pl.BlockSpec((D, D), lambda i: (0, 0)),
            pl.BlockSpec((D,), lambda i: (0,)),
        ],
        out_specs=pl.BlockSpec((_ROWBLK, D), lambda i: (i, 0)),
        out_shape=jax.ShapeDtypeStruct((N, D), jnp.float32),
    )(tab, acc, degp, ws, wn, b)


def _deg_body(dst_cb, dst_cl, degp_u_out, degp_i_out, idx_v, deg_local):
    c = lax.axis_index("c")
    s = lax.axis_index("s")
    zv = jnp.zeros((16,), jnp.float32)
    ones = jnp.ones((16,), jnp.float32)

    def _zero(i, carry):
        deg_local[pl.ds(i * 16, 16)] = zv
        return carry

    lax.fori_loop(0, NP // 16, _zero, 0)

    def _run(dsts, out):
        base0 = s * EPT

        def _step(i, carry):
            pltpu.sync_copy(dsts.at[pl.ds(base0 + i * CH, CH)], idx_v)
            for k in range(CH // 16):
                idx16 = idx_v[pl.ds(k * 16, 16)]
                plsc.addupdate_scatter(deg_local, [idx16], ones)
            return carry

        lax.fori_loop(0, NFULL, _step, 0)

        if TAIL:
            pltpu.sync_copy(dsts.at[pl.ds(base0 + NFULL * CH, TAIL)],
                            idx_v.at[pl.ds(0, TAIL)])
            for k in range(TAIL // 16):
                idx16 = idx_v[pl.ds(k * 16, 16)]
                plsc.addupdate_scatter(deg_local, [idx16], ones)

        pltpu.sync_copy(deg_local, out.at[pl.ds(s * NP, NP)])

    @pl.when(c == 0)
    def _():
        _run(dst_cb, degp_u_out)

    @pl.when(c == 1)
    def _():
        _run(dst_cl, degp_i_out)


@functools.lru_cache(maxsize=None)
def _deg_kernel():
    return pl.kernel(
        _deg_body,
        out_type=[
            jax.ShapeDtypeStruct((NS * NP,), jnp.float32),
            jax.ShapeDtypeStruct((NS * NP,), jnp.float32),
        ],
        mesh=plsc.VectorSubcoreMesh(core_axis_name="c", subcore_axis_name="s"),
        scratch_types=[
            pltpu.VMEM((CH,), jnp.int32),
            pltpu.VMEM((NP,), jnp.float32),
        ],
        compiler_params=pltpu.CompilerParams(needs_layout_passes=False),
    )


def _degrees(dst_cb, dst_cl):
    degp_u, degp_i = _deg_kernel()(dst_cb, dst_cl)
    return (degp_u.reshape(NS, NP)[:, :N].T, degp_i.reshape(NS, NP)[:, :N].T)


NBUF = 4
NGRP = NFULL // NBUF


def _seg_body(tab_cb, tab_cl, src_cb, dst_cb, src_cl, dst_cl,
              acc_u_out, acc_i_out, *scr):
    src_b = scr[0:NBUF]
    dst_b = scr[NBUF:2 * NBUF]
    rows_b = scr[2 * NBUF:3 * NBUF]
    src_t, dst_t, rows_t, acc_sh = scr[3 * NBUF:3 * NBUF + 4]
    isem = scr[3 * NBUF + 4:4 * NBUF + 4]
    gsem = scr[4 * NBUF + 4:5 * NBUF + 4]
    ssem = scr[5 * NBUF + 4:6 * NBUF + 4]
    tsem = scr[6 * NBUF + 4]

    c = lax.axis_index("c")
    s = lax.axis_index("s")

    zv = jnp.zeros((16,), jnp.float32)

    def _zero_row(r, carry):
        for j in range(D // 16):
            rows_b[0][r, pl.ds(j * 16, 16)] = zv
        return carry

    lax.fori_loop(0, RB, _zero_row, 0)

    def _run(tab, srcs, dsts, out):
        for k in range(RPT // RB):
            pltpu.sync_copy(rows_b[0],
                            acc_sh.at[pl.ds(s * RPT + k * RB, RB)])
        plsc.subcore_barrier()

        base0 = s * EPT

        def _fire_idx(g, b):
            base = base0 + (g * NBUF + b) * CH
            pltpu.async_copy(srcs.at[pl.ds(base, CH)], src_b[b], isem[b])
            pltpu.async_copy(dsts.at[pl.ds(base, CH)], dst_b[b], isem[b])

        for b in range(NBUF):
            _fire_idx(0, b)

        def _group(g, carry):
            gdesc = []
            for b in range(NBUF):
                pltpu.make_async_copy(srcs.at[pl.ds(0, CH)], src_b[b],
                                      isem[b]).wait()
                pltpu.make_async_copy(dsts.at[pl.ds(0, CH)], dst_b[b],
                                      isem[b]).wait()
                gdesc.append(
                    pltpu.async_copy(tab.at[src_b[b]], rows_b[b], gsem[b]))
            sdesc = []
            for b in range(NBUF):
                gdesc[b].wait()
                sdesc.append(
                    pltpu.async_copy(rows_b[b], acc_sh.at[dst_b[b]],
                                     ssem[b], add=True))
            for b in range(NBUF):
                sdesc[b].wait()

                @pl.when(g < NGRP - 1)
                def _():
                    _fire_idx(g + 1, b)

            return carry

        lax.fori_loop(0, NGRP, _group, 0)

        if TAIL:
            tb = base0 + NFULL * CH
            pltpu.sync_copy(srcs.at[pl.ds(tb, TAIL)], src_t)
            pltpu.sync_copy(dsts.at[pl.ds(tb, TAIL)], dst_t)
            pltpu.async_copy(tab.at[src_t], rows_t, tsem).wait()
            pltpu.sync_copy(rows_t, acc_sh.at[dst_t], add=True)

        plsc.subcore_barrier()
        pltpu.sync_copy(acc_sh.at[pl.ds(s * RPT, RPT)],
                        out.at[pl.ds(s * RPT, RPT)])

    @pl.when(c == 0)
    def _():
        _run(tab_cb, src_cb, dst_cb, acc_u_out)

    @pl.when(c == 1)
    def _():
        _run(tab_cl, src_cl, dst_cl, acc_i_out)


@functools.lru_cache(maxsize=None)
def _seg_layer_kernel():
    return pl.kernel(
        _seg_body,
        out_type=[
            jax.ShapeDtypeStruct((NP, D), jnp.float32),
            jax.ShapeDtypeStruct((NP, D), jnp.float32),
        ],
        mesh=plsc.VectorSubcoreMesh(core_axis_name="c", subcore_axis_name="s"),
        scratch_types=(
            [pltpu.VMEM((CH,), jnp.int32) for _ in range(NBUF)]
            + [pltpu.VMEM((CH,), jnp.int32) for _ in range(NBUF)]
            + [pltpu.VMEM((CH, D), jnp.float32) for _ in range(NBUF)]
            + [
                pltpu.VMEM((TAIL,), jnp.int32),
                pltpu.VMEM((TAIL,), jnp.int32),
                pltpu.VMEM((TAIL, D), jnp.float32),
                pltpu.VMEM_SHARED((NP, D), jnp.float32),
            ]
            + [pltpu.SemaphoreType.DMA for _ in range(3 * NBUF + 1)]
        ),
    )


def _seg_layer(tab_cb, tab_cl, src_cb, dst_cb, src_cl, dst_cl):
    acc_u, acc_i = _seg_layer_kernel()(tab_cb, tab_cl, src_cb, dst_cb,
                                       src_cl, dst_cl)
    return acc_u[:N], acc_i[:N]


def kernel(x_user, x_item, ei_clicks, ei_clicked_by, Wp_user, bp_user,
           Wp_item, bp_item, Ws0_clicks, Wn0_clicks, b0_clicks, Ws0_cb,
           Wn0_cb, b0_cb, Ws1_clicks, Wn1_clicks, b1_clicks, Ws1_cb,
           Wn1_cb, b1_cb):
    src_cb = ei_clicked_by[0]
    dst_cb = ei_clicked_by[1]
    src_cl = ei_clicks[0]
    dst_cl = ei_clicks[1]

    degp_u, degp_i = _degrees(dst_cb, dst_cl)

    hu = _proj(x_user, Wp_user, bp_user)
    hi = _proj(x_item, Wp_item, bp_item)

    acc_u, acc_i = _seg_layer(hi, hu, src_cb, dst_cb, src_cl, dst_cl)
    hu = _combine(hu, acc_u, degp_u, Ws0_cb, Wn0_cb, b0_cb, final=False)
    hi = _combine(hi, acc_i, degp_i, Ws0_clicks, Wn0_clicks, b0_clicks,
                  final=False)

    acc_u, acc_i = _seg_layer(hi, hu, src_cb, dst_cb, src_cl, dst_cl)
    h_u = _combine(hu, acc_u, degp_u, Ws1_cb, Wn1_cb, b1_cb, final=True)
    h_i = _combine(hi, acc_i, degp_i, Ws1_clicks, Wn1_clicks, b1_clicks,
                   final=True)
    return (h_u, h_i)


# speedup vs baseline: 6.9119x; 1.2017x over previous
import functools

import jax
import jax.numpy as jnp
from jax import lax
from jax.experimental import pallas as pl
from jax.experimental.pallas import tpu as pltpu
from jax.experimental.pallas import tpu_sc as plsc

N = 10000
E = 320000
D = 128
NS = 16
EPT = E // NS
CH = 64
NFULL = EPT // CH
TAIL = EPT - NFULL * CH
NP = 10240
RPT = NP // NS
RB = CH

_ROWBLK = 1000


def _proj_body(x_ref, w_ref, b_ref, o_ref):
    h = jnp.dot(x_ref[...], w_ref[...], preferred_element_type=jnp.float32,
                precision=lax.Precision.HIGHEST)
    o_ref[...] = jnp.maximum(h + b_ref[...][None, :], 0.0)


def _proj(x, w, b):
    return pl.pallas_call(
        _proj_body,
        grid=(N // _ROWBLK,),
        in_specs=[
            pl.BlockSpec((_ROWBLK, D), lambda i: (i, 0)),
            pl.BlockSpec((D, D), lambda i: (0, 0)),
            pl.BlockSpec((D,), lambda i: (0,)),
        ],
        out_specs=pl.BlockSpec((_ROWBLK, D), lambda i: (i, 0)),
        out_shape=jax.ShapeDtypeStruct((N, D), jnp.float32),
    )(x, w, b)


def _combine_body(final, tab_ref, acc_ref, degp_ref, ws_ref, wn_ref, b_ref,
                  o_ref):
    deg = jnp.sum(degp_ref[...], axis=1)
    hn = acc_ref[...] / jnp.maximum(deg, 1.0)[:, None]
    z = (jnp.dot(tab_ref[...], ws_ref[...], preferred_element_type=jnp.float32,
                 precision=lax.Precision.HIGHEST)
         + jnp.dot(hn, wn_ref[...], preferred_element_type=jnp.float32,
                   precision=lax.Precision.HIGHEST)
         + b_ref[...][None, :])
    z = jnp.maximum(z, 0.0)
    if final:
        nrm = jnp.sqrt(jnp.sum(z * z, axis=1, keepdims=True))
        z = z / jnp.maximum(nrm, 1e-12)
    o_ref[...] = z


def _combine(tab, acc, degp, ws, wn, b, final):
    return pl.pallas_call(
        functools.partial(_combine_body, final),
        grid=(N // _ROWBLK,),
        in_specs=[
            pl.BlockSpec((_ROWBLK, D), lambda i: (i, 0)),
            pl.BlockSpec((_ROWBLK, D), lambda i: (i, 0)),
            pl.BlockSpec((_ROWBLK, NS), lambda i: (i, 0)),
            pl.BlockSpec((D, D), lambda i: (0, 0)),
            pl.BlockSpec((D, D), lambda i: (0, 0)),
            pl.BlockSpec((D,), lambda i: (0,)),
        ],
        out_specs=pl.BlockSpec((_ROWBLK, D), lambda i: (i, 0)),
        out_shape=jax.ShapeDtypeStruct((N, D), jnp.float32),
    )(tab, acc, degp, ws, wn, b)


NBUF = 4
NGRP = NFULL // NBUF


def _seg_body(with_deg, tab_cb, tab_cl, src_cb, dst_cb, src_cl, dst_cl,
              *rest):
    nout = 4 if with_deg else 2
    outs = rest[:nout]
    scr = rest[nout:]
    acc_u_out, acc_i_out = outs[0], outs[1]
    degp_u_out = outs[2] if with_deg else None
    degp_i_out = outs[3] if with_deg else None

    src_b = scr[0:NBUF]
    dst_b = scr[NBUF:2 * NBUF]
    rows_b = scr[2 * NBUF:3 * NBUF]
    src_t, dst_t, acc_sh = scr[3 * NBUF:3 * NBUF + 3]
    k = 3 * NBUF + 3
    deg_local = scr[k] if with_deg else None
    k += 1 if with_deg else 0
    isem = scr[k:k + NBUF]
    gsem = scr[k + NBUF:k + 2 * NBUF]
    ssem = scr[k + 2 * NBUF:k + 3 * NBUF]
    tsem = scr[k + 3 * NBUF]

    c = lax.axis_index("c")
    s = lax.axis_index("s")

    zv = jnp.zeros((16,), jnp.float32)
    ones = jnp.ones((16,), jnp.float32)

    def _zero_row(r, carry):
        for j in range(D // 16):
            rows_b[0][r, pl.ds(j * 16, 16)] = zv
        return carry

    lax.fori_loop(0, RB, _zero_row, 0)

    if with_deg:
        def _zero_deg(i, carry):
            deg_local[pl.ds(i * 16, 16)] = zv
            return carry

        lax.fori_loop(0, NP // 16, _zero_deg, 0)

    def _hist(idx_ref, count):
        for j in range(count // 16):
            idx16 = idx_ref[pl.ds(j * 16, 16)]
            plsc.addupdate_scatter(deg_local, [idx16], ones)

    def _run(tab, srcs, dsts, out, degp_out):
        for j in range(RPT // RB):
            pltpu.sync_copy(rows_b[0],
                            acc_sh.at[pl.ds(s * RPT + j * RB, RB)])
        plsc.subcore_barrier()

        base0 = s * EPT

        def _fire_idx(g, b):
            base = base0 + (g * NBUF + b) * CH
            pltpu.async_copy(srcs.at[pl.ds(base, CH)], src_b[b], isem[b])
            pltpu.async_copy(dsts.at[pl.ds(base, CH)], dst_b[b], isem[b])

        for b in range(NBUF):
            _fire_idx(0, b)

        def _group(g, carry):
            gdesc = []
            for b in range(NBUF):
                pltpu.make_async_copy(srcs.at[pl.ds(0, CH)], src_b[b],
                                      isem[b]).wait()
                pltpu.make_async_copy(dsts.at[pl.ds(0, CH)], dst_b[b],
                                      isem[b]).wait()
                gdesc.append(
                    pltpu.async_copy(tab.at[src_b[b]], rows_b[b], gsem[b]))
            sdesc = []
            for b in range(NBUF):
                gdesc[b].wait()
                sdesc.append(
                    pltpu.async_copy(rows_b[b], acc_sh.at[dst_b[b]],
                                     ssem[b], add=True))
                if with_deg:
                    _hist(dst_b[b], CH)
            for b in range(NBUF):
                sdesc[b].wait()

                @pl.when(g < NGRP - 1)
                def _():
                    _fire_idx(g + 1, b)

            return carry

        lax.fori_loop(0, NGRP, _group, 0)

        if TAIL:
            tb = base0 + NFULL * CH
            pltpu.sync_copy(srcs.at[pl.ds(tb, TAIL)], src_t)
            pltpu.sync_copy(dsts.at[pl.ds(tb, TAIL)], dst_t)
            rows_tail = rows_b[0].at[pl.ds(0, TAIL)]
            pltpu.async_copy(tab.at[src_t], rows_tail, tsem).wait()
            pltpu.sync_copy(rows_tail, acc_sh.at[dst_t], add=True)
            if with_deg:
                _hist(dst_t, TAIL)

        if with_deg:
            pltpu.sync_copy(deg_local, degp_out.at[pl.ds(s * NP, NP)])

        plsc.subcore_barrier()
        pltpu.sync_copy(acc_sh.at[pl.ds(s * RPT, RPT)],
                        out.at[pl.ds(s * RPT, RPT)])

    @pl.when(c == 0)
    def _():
        _run(tab_cb, src_cb, dst_cb, acc_u_out, degp_u_out)

    @pl.when(c == 1)
    def _():
        _run(tab_cl, src_cl, dst_cl, acc_i_out, degp_i_out)


@functools.lru_cache(maxsize=None)
def _seg_layer_kernel(with_deg):
    out_type = [
        jax.ShapeDtypeStruct((NP, D), jnp.float32),
        jax.ShapeDtypeStruct((NP, D), jnp.float32),
    ]
    if with_deg:
        out_type += [
            jax.ShapeDtypeStruct((NS * NP,), jnp.float32),
            jax.ShapeDtypeStruct((NS * NP,), jnp.float32),
        ]
    return pl.kernel(
        functools.partial(_seg_body, with_deg),
        out_type=out_type,
        mesh=plsc.VectorSubcoreMesh(core_axis_name="c", subcore_axis_name="s"),
        scratch_types=(
            [pltpu.VMEM((CH,), jnp.int32) for _ in range(NBUF)]
            + [pltpu.VMEM((CH,), jnp.int32) for _ in range(NBUF)]
            + [pltpu.VMEM((CH, D), jnp.float32) for _ in range(NBUF)]
            + [
                pltpu.VMEM((TAIL,), jnp.int32),
                pltpu.VMEM((TAIL,), jnp.int32),
                pltpu.VMEM_SHARED((NP, D), jnp.float32),
            ]
            + ([pltpu.VMEM((NP,), jnp.float32)] if with_deg else [])
            + [pltpu.SemaphoreType.DMA for _ in range(3 * NBUF + 1)]
        ),
        compiler_params=pltpu.CompilerParams(needs_layout_passes=False),
    )


def _seg_layer0(tab_cb, tab_cl, src_cb, dst_cb, src_cl, dst_cl):
    acc_u, acc_i, degp_u, degp_i = _seg_layer_kernel(True)(
        tab_cb, tab_cl, src_cb, dst_cb, src_cl, dst_cl)
    return (acc_u[:N], acc_i[:N],
            degp_u.reshape(NS, NP)[:, :N].T, degp_i.reshape(NS, NP)[:, :N].T)


def _seg_layer(tab_cb, tab_cl, src_cb, dst_cb, src_cl, dst_cl):
    acc_u, acc_i = _seg_layer_kernel(False)(tab_cb, tab_cl, src_cb, dst_cb,
                                            src_cl, dst_cl)
    return acc_u[:N], acc_i[:N]


def kernel(x_user, x_item, ei_clicks, ei_clicked_by, Wp_user, bp_user,
           Wp_item, bp_item, Ws0_clicks, Wn0_clicks, b0_clicks, Ws0_cb,
           Wn0_cb, b0_cb, Ws1_clicks, Wn1_clicks, b1_clicks, Ws1_cb,
           Wn1_cb, b1_cb):
    src_cb = ei_clicked_by[0]
    dst_cb = ei_clicked_by[1]
    src_cl = ei_clicks[0]
    dst_cl = ei_clicks[1]

    hu = _proj(x_user, Wp_user, bp_user)
    hi = _proj(x_item, Wp_item, bp_item)

    acc_u, acc_i, degp_u, degp_i = _seg_layer0(hi, hu, src_cb, dst_cb,
                                               src_cl, dst_cl)
    hu = _combine(hu, acc_u, degp_u, Ws0_cb, Wn0_cb, b0_cb, final=False)
    hi = _combine(hi, acc_i, degp_i, Ws0_clicks, Wn0_clicks, b0_clicks,
                  final=False)

    acc_u, acc_i = _seg_layer(hi, hu, src_cb, dst_cb, src_cl, dst_cl)
    h_u = _combine(hu, acc_u, degp_u, Ws1_cb, Wn1_cb, b1_cb, final=True)
    h_i = _combine(hi, acc_i, degp_i, Ws1_clicks, Wn1_clicks, b1_clicks,
                   final=True)
    return (h_u, h_i)
